# baseline (speedup 1.0000x reference)
import jax
import jax.numpy as jnp
from jax import lax
from jax.experimental import pallas as pl
from jax.experimental.pallas import tpu as pltpu
from jax.experimental.pallas import tpu_sc as plsc

N = 100000
E = 3200000
NC = 2
NS = 16
NW = NC * NS
EW = E // NW
C = 4000
NCH = EW // C
C1 = 10000
NCH1 = EW // C1
EW1 = EW
NPAD = 100096
TS = NPAD // NS
TS_A = 4000
TS_B = TS - TS_A
R = NPAD // 128


def _zero_vec(buf, n):
    def z(i, _):
        buf[pl.ds(i * 16, 16)] = jnp.zeros((16,), jnp.float32)
        return 0
    lax.fori_loop(0, n // 16, z, 0)


def _reg_gather(x_loc, idx_v, out_v):
    def g(k, _):
        base = k * 32
        for u in range(2):
            o = base + u * 16
            out_v[pl.ds(o, 16)] = plsc.load_gather(
                x_loc, (idx_v[pl.ds(o, 16)],))
        return 0
    lax.fori_loop(0, C // 32, g, 0)


def _sweep_deg_body(x_hbm, src_hbm, dst_hbm, s_out, d_out,
                    x_sh, s_sh, d_sh, node_buf,
                    src_v0, src_v1, dst_v0, dst_v1, val_v0, val_v1, one_v,
                    sem_s, sem_d, sem_g, sem_o, sem_v):
    src_v = [src_v0, src_v1]
    dst_v = [dst_v0, dst_v1]
    val_v = [val_v0, val_v1]

    c = lax.axis_index("c")
    s = lax.axis_index("s")
    wid = c * NS + s
    base = wid * EW1

    ld_s = [None, None]
    ld_d = [None, None]
    ld_s[0] = pltpu.async_copy(src_hbm.at[pl.ds(base, C1)], src_v[0], sem_s)
    ld_d[0] = pltpu.async_copy(dst_hbm.at[pl.ds(base, C1)], dst_v[0], sem_d)

    _zero_vec(node_buf, TS)
    pltpu.sync_copy(node_buf, s_sh.at[pl.ds(s * TS, TS)])
    pltpu.sync_copy(node_buf, d_sh.at[pl.ds(s * TS, TS)])

    def o(i, _):
        one_v[pl.ds(i * 16, 16)] = jnp.full((16,), 1.0, jnp.float32)
        return 0
    lax.fori_loop(0, C1 // 16, o, 0)

    pltpu.sync_copy(x_hbm.at[pl.ds(s * TS, TS)], node_buf)
    pltpu.sync_copy(node_buf, x_sh.at[pl.ds(s * TS, TS)])

    plsc.subcore_barrier()

    sc_o = [None, None]
    sc_v = [None, None]
    for i in range(NCH1):
        b = i % 2
        nb = 1 - b
        ld_s[b].wait()
        ld_d[b].wait()
        g = pltpu.async_copy(x_sh.at[src_v[b]], val_v[b], sem_g)
        if sc_o[nb] is not None:
            sc_o[nb].wait()
            sc_o[nb] = None
        if sc_v[nb] is not None:
            sc_v[nb].wait()
            sc_v[nb] = None
        if i + 1 < NCH1:
            off = base + (i + 1) * C1
            ld_s[nb] = pltpu.async_copy(src_hbm.at[pl.ds(off, C1)],
                                        src_v[nb], sem_s)
            ld_d[nb] = pltpu.async_copy(dst_hbm.at[pl.ds(off, C1)],
                                        dst_v[nb], sem_d)
        sc_o[b] = pltpu.async_copy(one_v, d_sh.at[src_v[b]], sem_o, add=True)
        g.wait()
        sc_v[b] = pltpu.async_copy(val_v[b], s_sh.at[dst_v[b]], sem_v, add=True)
    for b in range(2):
        if sc_o[b] is not None:
            sc_o[b].wait()
        if sc_v[b] is not None:
            sc_v[b].wait()

    plsc.subcore_barrier()

    pltpu.sync_copy(s_sh.at[pl.ds(s * TS, TS)], node_buf)
    pltpu.sync_copy(node_buf, s_out.at[pl.ds(c * NPAD + s * TS, TS)])
    pltpu.sync_copy(d_sh.at[pl.ds(s * TS, TS)], node_buf)
    pltpu.sync_copy(node_buf, d_out.at[pl.ds(c * NPAD + s * TS, TS)])


def _sweep_body(x_hbm, src_hbm, dst_hbm, s_out,
                x_loc, s_sh,
                src_v0, src_v1, dst_v0, dst_v1, val_v0, val_v1,
                sem_s, sem_d, sem_v):
    src_v = [src_v0, src_v1]
    dst_v = [dst_v0, dst_v1]
    val_v = [val_v0, val_v1]

    c = lax.axis_index("c")
    s = lax.axis_index("s")
    wid = c * NS + s
    base = wid * EW

    ld_s = [None, None]
    ld_d = [None, None]
    ld_s[0] = pltpu.async_copy(src_hbm.at[pl.ds(base, C)], src_v[0], sem_s)
    ld_d[0] = pltpu.async_copy(dst_hbm.at[pl.ds(base, C)], dst_v[0], sem_d)
    ld_x = pltpu.async_copy(x_hbm, x_loc, sem_v)

    _zero_vec(val_v[0], C)
    pltpu.sync_copy(val_v[0].at[pl.ds(0, TS_A)],
                    s_sh.at[pl.ds(s * TS, TS_A)])
    pltpu.sync_copy(val_v[0].at[pl.ds(0, TS_B)],
                    s_sh.at[pl.ds(s * TS + TS_A, TS_B)])
    ld_x.wait()

    plsc.subcore_barrier()

    sc_v = [None, None]
    for i in range(NCH):
        b = i % 2
        nb = 1 - b
        ld_s[b].wait()
        ld_d[b].wait()
        _reg_gather(x_loc, src_v[b], val_v[b])
        if sc_v[nb] is not None:
            sc_v[nb].wait()
            sc_v[nb] = None
        if i + 1 < NCH:
            off = base + (i + 1) * C
            ld_s[nb] = pltpu.async_copy(src_hbm.at[pl.ds(off, C)],
                                        src_v[nb], sem_s)
            ld_d[nb] = pltpu.async_copy(dst_hbm.at[pl.ds(off, C)],
                                        dst_v[nb], sem_d)
        sc_v[b] = pltpu.async_copy(val_v[b], s_sh.at[dst_v[b]], sem_v, add=True)
    for b in range(2):
        if sc_v[b] is not None:
            sc_v[b].wait()

    plsc.subcore_barrier()

    pltpu.sync_copy(s_sh.at[pl.ds(s * TS, TS_A)], val_v[0])
    pltpu.sync_copy(s_sh.at[pl.ds(s * TS + TS_A, TS_B)],
                    val_v[1].at[pl.ds(0, TS_B)])
    pltpu.sync_copy(val_v[0], s_out.at[pl.ds(c * NPAD + s * TS, TS_A)])
    pltpu.sync_copy(val_v[1].at[pl.ds(0, TS_B)],
                    s_out.at[pl.ds(c * NPAD + s * TS + TS_A, TS_B)])


_MESH = plsc.VectorSubcoreMesh(core_axis_name="c", subcore_axis_name="s",
                               num_cores=NC, num_subcores=NS)

_sweep_deg = pl.kernel(
    _sweep_deg_body,
    out_type=(jax.ShapeDtypeStruct((NC * NPAD,), jnp.float32),
              jax.ShapeDtypeStruct((NC * NPAD,), jnp.float32)),
    mesh=_MESH,
    scratch_types=[
        pltpu.VMEM_SHARED((NPAD,), jnp.float32),
        pltpu.VMEM_SHARED((NPAD,), jnp.float32),
        pltpu.VMEM_SHARED((NPAD,), jnp.float32),
        pltpu.VMEM((TS,), jnp.float32),
        pltpu.VMEM((C1,), jnp.int32),
        pltpu.VMEM((C1,), jnp.int32),
        pltpu.VMEM((C1,), jnp.int32),
        pltpu.VMEM((C1,), jnp.int32),
        pltpu.VMEM((C1,), jnp.float32),
        pltpu.VMEM((C1,), jnp.float32),
        pltpu.VMEM((C1,), jnp.float32),
        pltpu.SemaphoreType.DMA,
        pltpu.SemaphoreType.DMA,
        pltpu.SemaphoreType.DMA,
        pltpu.SemaphoreType.DMA,
        pltpu.SemaphoreType.DMA,
    ],
    name="dgmrf_sweep_deg",
)

_sweep = pl.kernel(
    _sweep_body,
    out_type=jax.ShapeDtypeStruct((NC * NPAD,), jnp.float32),
    mesh=_MESH,
    scratch_types=[
        pltpu.VMEM((NPAD,), jnp.float32),
        pltpu.VMEM_SHARED((NPAD,), jnp.float32),
        pltpu.VMEM((C,), jnp.int32),
        pltpu.VMEM((C,), jnp.int32),
        pltpu.VMEM((C,), jnp.int32),
        pltpu.VMEM((C,), jnp.int32),
        pltpu.VMEM((C,), jnp.float32),
        pltpu.VMEM((C,), jnp.float32),
        pltpu.SemaphoreType.DMA,
        pltpu.SemaphoreType.DMA,
        pltpu.SemaphoreType.DMA,
    ],
    name="dgmrf_sweep",
    compiler_params=pltpu.CompilerParams(needs_layout_passes=False),
)


def _mid_body(g_ref, a1_ref, a2_ref, b_ref, aw_ref,
              x_ref, d0_ref, d1_ref, s0_ref, s1_ref, x1_ref, logd_ref):
    deg = jnp.maximum(d0_ref[...] + d1_ref[...], 1.0)
    logd = jnp.log(deg)
    dp = 1.0 / (1.0 + jnp.exp(-g_ref[0]))
    sw = jnp.exp(a1_ref[0])
    nw = sw * jnp.tanh(a2_ref[0])
    agg = s0_ref[...] + s1_ref[...]
    y = (sw * x_ref[...] * jnp.exp(dp * logd)
         + nw * jnp.exp((dp - 1.0) * logd) * agg + b_ref[0])
    w = jax.nn.softplus(aw_ref[0])
    x1_ref[...] = jnp.where(y >= 0.0, y, w * y)
    logd_ref[...] = logd


def _fin_body(g_ref, a1_ref, a2_ref, b_ref,
              x_ref, logd_ref, s0_ref, s1_ref, o_ref):
    logd = logd_ref[...]
    dp = 1.0 / (1.0 + jnp.exp(-g_ref[0]))
    sw = jnp.exp(a1_ref[0])
    nw = sw * jnp.tanh(a2_ref[0])
    agg = s0_ref[...] + s1_ref[...]
    o_ref[...] = (sw * x_ref[...] * jnp.exp(dp * logd)
                  + nw * jnp.exp((dp - 1.0) * logd) * agg + b_ref[0])


_SMEM1 = pl.BlockSpec(memory_space=pltpu.SMEM)
_VSPEC = pl.BlockSpec(memory_space=pltpu.VMEM)

_mid = pl.pallas_call(
    _mid_body,
    out_shape=(jax.ShapeDtypeStruct((R, 128), jnp.float32),
               jax.ShapeDtypeStruct((R, 128), jnp.float32)),
    in_specs=[_SMEM1] * 5 + [_VSPEC] * 5,
    out_specs=(_VSPEC, _VSPEC),
    name="dgmrf_mid",
)

_fin = pl.pallas_call(
    _fin_body,
    out_shape=jax.ShapeDtypeStruct((R, 128), jnp.float32),
    in_specs=[_SMEM1] * 4 + [_VSPEC] * 4,
    out_specs=_VSPEC,
    name="dgmrf_fin",
)


def kernel(x, edge_index, alpha1_0, alpha2_0, gamma_0, bias_0, act_weight_0,
           alpha1_1, alpha2_1, gamma_1, bias_1):
    x0 = jnp.pad(x.reshape(N), (0, NPAD - N))
    src = edge_index[0]
    dst = edge_index[1]

    s0_par, deg_par = _sweep_deg(x0, src, dst)

    x1_2d, logd_2d = _mid(
        gamma_0, alpha1_0, alpha2_0, bias_0, act_weight_0,
        x0.reshape(R, 128),
        deg_par[:NPAD].reshape(R, 128), deg_par[NPAD:].reshape(R, 128),
        s0_par[:NPAD].reshape(R, 128), s0_par[NPAD:].reshape(R, 128))

    s1_par = _sweep(x1_2d.reshape(NPAD), src, dst)

    out_2d = _fin(
        gamma_1, alpha1_1, alpha2_1, bias_1,
        x1_2d, logd_2d,
        s1_par[:NPAD].reshape(R, 128), s1_par[NPAD:].reshape(R, 128))

    return out_2d.reshape(NPAD)[:N].reshape(N, 1)


# speedup vs baseline: 1.0697x; 1.0697x over previous
import jax
import jax.numpy as jnp
from jax import lax
from jax.experimental import pallas as pl
from jax.experimental.pallas import tpu as pltpu
from jax.experimental.pallas import tpu_sc as plsc

N = 100000
E = 3200000
NC = 2
NS = 16
NW = NC * NS
EW = E // NW
C = 4000
NCH = EW // C
C1 = 2000
NCH1 = EW // C1
EW1 = EW
ZP = (2000, 2000, 2000, 256)
NPAD = 100096
TS = NPAD // NS
TS_A = 4000
TS_B = TS - TS_A
R = NPAD // 128


def _zero_vec(buf, n):
    def z(i, _):
        buf[pl.ds(i * 16, 16)] = jnp.zeros((16,), jnp.float32)
        return 0
    lax.fori_loop(0, n // 16, z, 0)


def _reg_gather(x_loc, idx_v, out_v):
    def g(k, _):
        base = k * 32
        for u in range(2):
            o = base + u * 16
            out_v[pl.ds(o, 16)] = plsc.load_gather(
                x_loc, (idx_v[pl.ds(o, 16)],))
        return 0
    lax.fori_loop(0, C // 32, g, 0)


def _sweep_deg_body(x_hbm, src_hbm, dst_hbm, s_out, d_out,
                    x_loc, s_sh, d_sh,
                    src_v0, src_v1, dst_v0, dst_v1, val_v0, val_v1, one_v,
                    sem_s, sem_d, sem_o, sem_v):
    src_v = [src_v0, src_v1]
    dst_v = [dst_v0, dst_v1]
    val_v = [val_v0, val_v1]

    c = lax.axis_index("c")
    s = lax.axis_index("s")
    wid = c * NS + s
    base = wid * EW1

    ld_s = [None, None]
    ld_d = [None, None]
    ld_s[0] = pltpu.async_copy(src_hbm.at[pl.ds(base, C1)], src_v[0], sem_s)
    ld_d[0] = pltpu.async_copy(dst_hbm.at[pl.ds(base, C1)], dst_v[0], sem_d)
    ld_x = pltpu.async_copy(x_hbm, x_loc, sem_v)

    _zero_vec(val_v[0], C1)
    off = 0
    for z in ZP:
        pltpu.sync_copy(val_v[0].at[pl.ds(0, z)],
                        s_sh.at[pl.ds(s * TS + off, z)])
        pltpu.sync_copy(val_v[0].at[pl.ds(0, z)],
                        d_sh.at[pl.ds(s * TS + off, z)])
        off += z

    def o(i, _):
        one_v[pl.ds(i * 16, 16)] = jnp.full((16,), 1.0, jnp.float32)
        return 0
    lax.fori_loop(0, C1 // 16, o, 0)
    ld_x.wait()

    plsc.subcore_barrier()

    sc_o = [None, None]
    sc_v = [None, None]
    for i in range(NCH1):
        b = i % 2
        nb = 1 - b
        ld_s[b].wait()
        ld_d[b].wait()
        def g(k, _):
            val_v[b][pl.ds(k * 16, 16)] = plsc.load_gather(
                x_loc, (src_v[b][pl.ds(k * 16, 16)],))
            return 0
        lax.fori_loop(0, C1 // 16, g, 0)
        if sc_o[nb] is not None:
            sc_o[nb].wait()
            sc_o[nb] = None
        if sc_v[nb] is not None:
            sc_v[nb].wait()
            sc_v[nb] = None
        if i + 1 < NCH1:
            off2 = base + (i + 1) * C1
            ld_s[nb] = pltpu.async_copy(src_hbm.at[pl.ds(off2, C1)],
                                        src_v[nb], sem_s)
            ld_d[nb] = pltpu.async_copy(dst_hbm.at[pl.ds(off2, C1)],
                                        dst_v[nb], sem_d)
        sc_o[b] = pltpu.async_copy(one_v, d_sh.at[src_v[b]], sem_o, add=True)
        sc_v[b] = pltpu.async_copy(val_v[b], s_sh.at[dst_v[b]], sem_v, add=True)
    for b in range(2):
        if sc_o[b] is not None:
            sc_o[b].wait()
        if sc_v[b] is not None:
            sc_v[b].wait()

    plsc.subcore_barrier()

    off = 0
    for z in ZP:
        pltpu.sync_copy(s_sh.at[pl.ds(s * TS + off, z)], val_v[0].at[pl.ds(0, z)])
        pltpu.sync_copy(val_v[0].at[pl.ds(0, z)],
                        s_out.at[pl.ds(c * NPAD + s * TS + off, z)])
        pltpu.sync_copy(d_sh.at[pl.ds(s * TS + off, z)], val_v[1].at[pl.ds(0, z)])
        pltpu.sync_copy(val_v[1].at[pl.ds(0, z)],
                        d_out.at[pl.ds(c * NPAD + s * TS + off, z)])
        off += z


def _sweep_body(x_hbm, src_hbm, dst_hbm, s_out,
                x_loc, s_sh,
                src_v0, src_v1, dst_v0, dst_v1, val_v0, val_v1,
                sem_s, sem_d, sem_v):
    src_v = [src_v0, src_v1]
    dst_v = [dst_v0, dst_v1]
    val_v = [val_v0, val_v1]

    c = lax.axis_index("c")
    s = lax.axis_index("s")
    wid = c * NS + s
    base = wid * EW

    ld_s = [None, None]
    ld_d = [None, None]
    ld_s[0] = pltpu.async_copy(src_hbm.at[pl.ds(base, C)], src_v[0], sem_s)
    ld_d[0] = pltpu.async_copy(dst_hbm.at[pl.ds(base, C)], dst_v[0], sem_d)
    ld_x = pltpu.async_copy(x_hbm, x_loc, sem_v)

    _zero_vec(val_v[0], C)
    pltpu.sync_copy(val_v[0].at[pl.ds(0, TS_A)],
                    s_sh.at[pl.ds(s * TS, TS_A)])
    pltpu.sync_copy(val_v[0].at[pl.ds(0, TS_B)],
                    s_sh.at[pl.ds(s * TS + TS_A, TS_B)])
    ld_x.wait()

    plsc.subcore_barrier()

    sc_v = [None, None]
    for i in range(NCH):
        b = i % 2
        nb = 1 - b
        ld_s[b].wait()
        ld_d[b].wait()
        _reg_gather(x_loc, src_v[b], val_v[b])
        if sc_v[nb] is not None:
            sc_v[nb].wait()
            sc_v[nb] = None
        if i + 1 < NCH:
            off = base + (i + 1) * C
            ld_s[nb] = pltpu.async_copy(src_hbm.at[pl.ds(off, C)],
                                        src_v[nb], sem_s)
            ld_d[nb] = pltpu.async_copy(dst_hbm.at[pl.ds(off, C)],
                                        dst_v[nb], sem_d)
        sc_v[b] = pltpu.async_copy(val_v[b], s_sh.at[dst_v[b]], sem_v, add=True)
    for b in range(2):
        if sc_v[b] is not None:
            sc_v[b].wait()

    plsc.subcore_barrier()

    pltpu.sync_copy(s_sh.at[pl.ds(s * TS, TS_A)], val_v[0])
    pltpu.sync_copy(s_sh.at[pl.ds(s * TS + TS_A, TS_B)],
                    val_v[1].at[pl.ds(0, TS_B)])
    pltpu.sync_copy(val_v[0], s_out.at[pl.ds(c * NPAD + s * TS, TS_A)])
    pltpu.sync_copy(val_v[1].at[pl.ds(0, TS_B)],
                    s_out.at[pl.ds(c * NPAD + s * TS + TS_A, TS_B)])


_MESH = plsc.VectorSubcoreMesh(core_axis_name="c", subcore_axis_name="s",
                               num_cores=NC, num_subcores=NS)

_sweep_deg = pl.kernel(
    _sweep_deg_body,
    out_type=(jax.ShapeDtypeStruct((NC * NPAD,), jnp.float32),
              jax.ShapeDtypeStruct((NC * NPAD,), jnp.float32)),
    mesh=_MESH,
    scratch_types=[
        pltpu.VMEM((NPAD,), jnp.float32),
        pltpu.VMEM_SHARED((NPAD,), jnp.float32),
        pltpu.VMEM_SHARED((NPAD,), jnp.float32),
        pltpu.VMEM((C1,), jnp.int32),
        pltpu.VMEM((C1,), jnp.int32),
        pltpu.VMEM((C1,), jnp.int32),
        pltpu.VMEM((C1,), jnp.int32),
        pltpu.VMEM((C1,), jnp.float32),
        pltpu.VMEM((C1,), jnp.float32),
        pltpu.VMEM((C1,), jnp.float32),
        pltpu.SemaphoreType.DMA,
        pltpu.SemaphoreType.DMA,
        pltpu.SemaphoreType.DMA,
        pltpu.SemaphoreType.DMA,
    ],
    name="dgmrf_sweep_deg",
    compiler_params=pltpu.CompilerParams(needs_layout_passes=False),
)

_sweep = pl.kernel(
    _sweep_body,
    out_type=jax.ShapeDtypeStruct((NC * NPAD,), jnp.float32),
    mesh=_MESH,
    scratch_types=[
        pltpu.VMEM((NPAD,), jnp.float32),
        pltpu.VMEM_SHARED((NPAD,), jnp.float32),
        pltpu.VMEM((C,), jnp.int32),
        pltpu.VMEM((C,), jnp.int32),
        pltpu.VMEM((C,), jnp.int32),
        pltpu.VMEM((C,), jnp.int32),
        pltpu.VMEM((C,), jnp.float32),
        pltpu.VMEM((C,), jnp.float32),
        pltpu.SemaphoreType.DMA,
        pltpu.SemaphoreType.DMA,
        pltpu.SemaphoreType.DMA,
    ],
    name="dgmrf_sweep",
    compiler_params=pltpu.CompilerParams(needs_layout_passes=False),
)


def _mid_body(g_ref, a1_ref, a2_ref, b_ref, aw_ref,
              x_ref, d0_ref, d1_ref, s0_ref, s1_ref, x1_ref, logd_ref):
    deg = jnp.maximum(d0_ref[...] + d1_ref[...], 1.0)
    logd = jnp.log(deg)
    dp = 1.0 / (1.0 + jnp.exp(-g_ref[0]))
    sw = jnp.exp(a1_ref[0])
    nw = sw * jnp.tanh(a2_ref[0])
    agg = s0_ref[...] + s1_ref[...]
    y = (sw * x_ref[...] * jnp.exp(dp * logd)
         + nw * jnp.exp((dp - 1.0) * logd) * agg + b_ref[0])
    w = jax.nn.softplus(aw_ref[0])
    x1_ref[...] = jnp.where(y >= 0.0, y, w * y)
    logd_ref[...] = logd


def _fin_body(g_ref, a1_ref, a2_ref, b_ref,
              x_ref, logd_ref, s0_ref, s1_ref, o_ref):
    logd = logd_ref[...]
    dp = 1.0 / (1.0 + jnp.exp(-g_ref[0]))
    sw = jnp.exp(a1_ref[0])
    nw = sw * jnp.tanh(a2_ref[0])
    agg = s0_ref[...] + s1_ref[...]
    o_ref[...] = (sw * x_ref[...] * jnp.exp(dp * logd)
                  + nw * jnp.exp((dp - 1.0) * logd) * agg + b_ref[0])


_SMEM1 = pl.BlockSpec(memory_space=pltpu.SMEM)
_VSPEC = pl.BlockSpec(memory_space=pltpu.VMEM)

_mid = pl.pallas_call(
    _mid_body,
    out_shape=(jax.ShapeDtypeStruct((R, 128), jnp.float32),
               jax.ShapeDtypeStruct((R, 128), jnp.float32)),
    in_specs=[_SMEM1] * 5 + [_VSPEC] * 5,
    out_specs=(_VSPEC, _VSPEC),
    name="dgmrf_mid",
)

_fin = pl.pallas_call(
    _fin_body,
    out_shape=jax.ShapeDtypeStruct((R, 128), jnp.float32),
    in_specs=[_SMEM1] * 4 + [_VSPEC] * 4,
    out_specs=_VSPEC,
    name="dgmrf_fin",
)


def kernel(x, edge_index, alpha1_0, alpha2_0, gamma_0, bias_0, act_weight_0,
           alpha1_1, alpha2_1, gamma_1, bias_1):
    x0 = jnp.pad(x.reshape(N), (0, NPAD - N))
    src = edge_index[0]
    dst = edge_index[1]

    s0_par, deg_par = _sweep_deg(x0, src, dst)

    x1_2d, logd_2d = _mid(
        gamma_0, alpha1_0, alpha2_0, bias_0, act_weight_0,
        x0.reshape(R, 128),
        deg_par[:NPAD].reshape(R, 128), deg_par[NPAD:].reshape(R, 128),
        s0_par[:NPAD].reshape(R, 128), s0_par[NPAD:].reshape(R, 128))

    s1_par = _sweep(x1_2d.reshape(NPAD), src, dst)

    out_2d = _fin(
        gamma_1, alpha1_1, alpha2_1, bias_1,
        x1_2d, logd_2d,
        s1_par[:NPAD].reshape(R, 128), s1_par[NPAD:].reshape(R, 128))

    return out_2d.reshape(NPAD)[:N].reshape(N, 1)


# speedup vs baseline: 1.2110x; 1.1322x over previous
import jax
import jax.numpy as jnp
from jax import lax
from jax.experimental import pallas as pl
from jax.experimental.pallas import tpu as pltpu
from jax.experimental.pallas import tpu_sc as plsc

N = 100000
E = 3200000
NC = 2
NS = 16
NW = NC * NS
EW = E // NW
C = 2000
NCH = EW // C
NPAD = 100096
TS = NPAD // NS
ZP = (2000, 2000, 2000, 256)
R = NPAD // 128


def _zero_vec(buf, n):
    def z(i, _):
        buf[pl.ds(i * 16, 16)] = jnp.zeros((16,), jnp.float32)
        return 0
    lax.fori_loop(0, n // 16, z, 0)


def _reg_gather(x_loc, idx_v, out_v):
    def g(k, _):
        out_v[pl.ds(k * 16, 16)] = plsc.load_gather(
            x_loc, (idx_v[pl.ds(k * 16, 16)],))
        return 0
    lax.fori_loop(0, C // 16, g, 0)


def _sweep_deg_body(x_hbm, ei_hbm, s_out, d_out,
                    x_loc, s_sh, d_sh,
                    src_v0, src_v1, dst_v0, dst_v1, val_v0, val_v1, one_v,
                    sem_s, sem_d, sem_o, sem_v):
    src_v = [src_v0, src_v1]
    dst_v = [dst_v0, dst_v1]
    val_v = [val_v0, val_v1]

    c = lax.axis_index("c")
    s = lax.axis_index("s")
    wid = c * NS + s
    base = wid * EW

    ld_s = [None, None]
    ld_d = [None, None]
    ld_s[0] = pltpu.async_copy(ei_hbm.at[pl.ds(base, C)], src_v[0], sem_s)
    ld_d[0] = pltpu.async_copy(ei_hbm.at[pl.ds(E + base, C)], dst_v[0], sem_d)
    ld_x = pltpu.async_copy(x_hbm, x_loc, sem_v)

    _zero_vec(val_v[0], C)
    off = 0
    for z in ZP:
        pltpu.sync_copy(val_v[0].at[pl.ds(0, z)],
                        s_sh.at[pl.ds(s * TS + off, z)])
        pltpu.sync_copy(val_v[0].at[pl.ds(0, z)],
                        d_sh.at[pl.ds(s * TS + off, z)])
        off += z

    def o(i, _):
        one_v[pl.ds(i * 16, 16)] = jnp.full((16,), 1.0, jnp.float32)
        return 0
    lax.fori_loop(0, C // 16, o, 0)
    ld_x.wait()

    plsc.subcore_barrier()

    sc_o = [None, None]
    sc_v = [None, None]
    for i in range(NCH):
        b = i % 2
        nb = 1 - b
        ld_s[b].wait()
        ld_d[b].wait()
        _reg_gather(x_loc, src_v[b], val_v[b])
        if sc_o[nb] is not None:
            sc_o[nb].wait()
            sc_o[nb] = None
        if sc_v[nb] is not None:
            sc_v[nb].wait()
            sc_v[nb] = None
        if i + 1 < NCH:
            off2 = base + (i + 1) * C
            ld_s[nb] = pltpu.async_copy(ei_hbm.at[pl.ds(off2, C)],
                                        src_v[nb], sem_s)
            ld_d[nb] = pltpu.async_copy(ei_hbm.at[pl.ds(E + off2, C)],
                                        dst_v[nb], sem_d)
        sc_o[b] = pltpu.async_copy(one_v, d_sh.at[src_v[b]], sem_o, add=True)
        sc_v[b] = pltpu.async_copy(val_v[b], s_sh.at[dst_v[b]], sem_v, add=True)
    for b in range(2):
        if sc_o[b] is not None:
            sc_o[b].wait()
        if sc_v[b] is not None:
            sc_v[b].wait()

    plsc.subcore_barrier()

    off = 0
    for z in ZP:
        pltpu.sync_copy(s_sh.at[pl.ds(s * TS + off, z)], val_v[0].at[pl.ds(0, z)])
        pltpu.sync_copy(val_v[0].at[pl.ds(0, z)],
                        s_out.at[pl.ds(c * NPAD + s * TS + off, z)])
        pltpu.sync_copy(d_sh.at[pl.ds(s * TS + off, z)], val_v[1].at[pl.ds(0, z)])
        pltpu.sync_copy(val_v[1].at[pl.ds(0, z)],
                        d_out.at[pl.ds(c * NPAD + s * TS + off, z)])
        off += z


def _sweep_body(x_hbm, ei_hbm, s_out,
                x_loc, s_sh,
                src_v0, src_v1, src_v2, src_v3,
                dst_v0, dst_v1, dst_v2, dst_v3,
                val_v0, val_v1, val_v2, val_v3,
                sem_s, sem_d, sem_v):
    src_v = [src_v0, src_v1, src_v2, src_v3]
    dst_v = [dst_v0, dst_v1, dst_v2, dst_v3]
    val_v = [val_v0, val_v1, val_v2, val_v3]

    c = lax.axis_index("c")
    s = lax.axis_index("s")
    wid = c * NS + s
    base = wid * EW

    ld_s = [None, None, None, None]
    ld_d = [None, None, None, None]
    for j in range(2):
        ld_s[j] = pltpu.async_copy(ei_hbm.at[pl.ds(base + j * C, C)],
                                   src_v[j], sem_s)
        ld_d[j] = pltpu.async_copy(ei_hbm.at[pl.ds(E + base + j * C, C)],
                                   dst_v[j], sem_d)
    ld_x = pltpu.async_copy(x_hbm, x_loc, sem_v)

    _zero_vec(val_v[0], C)
    off = 0
    for z in ZP:
        pltpu.sync_copy(val_v[0].at[pl.ds(0, z)],
                        s_sh.at[pl.ds(s * TS + off, z)])
        off += z
    ld_x.wait()

    plsc.subcore_barrier()

    sc_v = [None, None, None, None]
    for i in range(NCH):
        b = i % 4
        ld_s[b].wait()
        ld_d[b].wait()
        _reg_gather(x_loc, src_v[b], val_v[b])
        pb = (i - 2) % 4
        if i >= 2 and sc_v[pb] is not None:
            sc_v[pb].wait()
            sc_v[pb] = None
        if i + 2 < NCH:
            off2 = base + (i + 2) * C
            nb = (i + 2) % 4
            ld_s[nb] = pltpu.async_copy(ei_hbm.at[pl.ds(off2, C)],
                                        src_v[nb], sem_s)
            ld_d[nb] = pltpu.async_copy(ei_hbm.at[pl.ds(E + off2, C)],
                                        dst_v[nb], sem_d)
        sc_v[b] = pltpu.async_copy(val_v[b], s_sh.at[dst_v[b]], sem_v, add=True)
    for b in range(4):
        if sc_v[b] is not None:
            sc_v[b].wait()

    plsc.subcore_barrier()

    off = 0
    for z in ZP:
        pltpu.sync_copy(s_sh.at[pl.ds(s * TS + off, z)], val_v[0].at[pl.ds(0, z)])
        pltpu.sync_copy(val_v[0].at[pl.ds(0, z)],
                        s_out.at[pl.ds(c * NPAD + s * TS + off, z)])
        off += z


_MESH = plsc.VectorSubcoreMesh(core_axis_name="c", subcore_axis_name="s",
                               num_cores=NC, num_subcores=NS)

_sweep_deg = pl.kernel(
    _sweep_deg_body,
    out_type=(jax.ShapeDtypeStruct((NC * NPAD,), jnp.float32),
              jax.ShapeDtypeStruct((NC * NPAD,), jnp.float32)),
    mesh=_MESH,
    scratch_types=[
        pltpu.VMEM((NPAD,), jnp.float32),
        pltpu.VMEM_SHARED((NPAD,), jnp.float32),
        pltpu.VMEM_SHARED((NPAD,), jnp.float32),
        pltpu.VMEM((C,), jnp.int32),
        pltpu.VMEM((C,), jnp.int32),
        pltpu.VMEM((C,), jnp.int32),
        pltpu.VMEM((C,), jnp.int32),
        pltpu.VMEM((C,), jnp.float32),
        pltpu.VMEM((C,), jnp.float32),
        pltpu.VMEM((C,), jnp.float32),
        pltpu.SemaphoreType.DMA,
        pltpu.SemaphoreType.DMA,
        pltpu.SemaphoreType.DMA,
        pltpu.SemaphoreType.DMA,
    ],
    name="dgmrf_sweep_deg",
    compiler_params=pltpu.CompilerParams(needs_layout_passes=False),
)

_sweep = pl.kernel(
    _sweep_body,
    out_type=jax.ShapeDtypeStruct((NC * NPAD,), jnp.float32),
    mesh=_MESH,
    scratch_types=[
        pltpu.VMEM((NPAD,), jnp.float32),
        pltpu.VMEM_SHARED((NPAD,), jnp.float32),
        pltpu.VMEM((C,), jnp.int32),
        pltpu.VMEM((C,), jnp.int32),
        pltpu.VMEM((C,), jnp.int32),
        pltpu.VMEM((C,), jnp.int32),
        pltpu.VMEM((C,), jnp.int32),
        pltpu.VMEM((C,), jnp.int32),
        pltpu.VMEM((C,), jnp.int32),
        pltpu.VMEM((C,), jnp.int32),
        pltpu.VMEM((C,), jnp.float32),
        pltpu.VMEM((C,), jnp.float32),
        pltpu.VMEM((C,), jnp.float32),
        pltpu.VMEM((C,), jnp.float32),
        pltpu.SemaphoreType.DMA,
        pltpu.SemaphoreType.DMA,
        pltpu.SemaphoreType.DMA,
    ],
    name="dgmrf_sweep",
    compiler_params=pltpu.CompilerParams(needs_layout_passes=False),
)


def _mid_body(g_ref, a1_ref, a2_ref, b_ref, aw_ref,
              x_ref, d_ref, s_ref, x1_ref, logd_ref):
    deg = jnp.maximum(d_ref[pl.ds(0, R), :] + d_ref[pl.ds(R, R), :], 1.0)
    logd = jnp.log(deg)
    dp = 1.0 / (1.0 + jnp.exp(-g_ref[0]))
    sw = jnp.exp(a1_ref[0])
    nw = sw * jnp.tanh(a2_ref[0])
    agg = s_ref[pl.ds(0, R), :] + s_ref[pl.ds(R, R), :]
    y = (sw * x_ref[...] * jnp.exp(dp * logd)
         + nw * jnp.exp((dp - 1.0) * logd) * agg + b_ref[0])
    w = jax.nn.softplus(aw_ref[0])
    x1_ref[...] = jnp.where(y >= 0.0, y, w * y)
    logd_ref[...] = logd


def _fin_body(g_ref, a1_ref, a2_ref, b_ref,
              x_ref, logd_ref, s_ref, o_ref):
    logd = logd_ref[...]
    dp = 1.0 / (1.0 + jnp.exp(-g_ref[0]))
    sw = jnp.exp(a1_ref[0])
    nw = sw * jnp.tanh(a2_ref[0])
    agg = s_ref[pl.ds(0, R), :] + s_ref[pl.ds(R, R), :]
    o_ref[...] = (sw * x_ref[...] * jnp.exp(dp * logd)
                  + nw * jnp.exp((dp - 1.0) * logd) * agg + b_ref[0])


_SMEM1 = pl.BlockSpec(memory_space=pltpu.SMEM)
_VSPEC = pl.BlockSpec(memory_space=pltpu.VMEM)

_mid = pl.pallas_call(
    _mid_body,
    out_shape=(jax.ShapeDtypeStruct((R, 128), jnp.float32),
               jax.ShapeDtypeStruct((R, 128), jnp.float32)),
    in_specs=[_SMEM1] * 5 + [_VSPEC] * 3,
    out_specs=(_VSPEC, _VSPEC),
    name="dgmrf_mid",
)

_fin = pl.pallas_call(
    _fin_body,
    out_shape=jax.ShapeDtypeStruct((R, 128), jnp.float32),
    in_specs=[_SMEM1] * 4 + [_VSPEC] * 3,
    out_specs=_VSPEC,
    name="dgmrf_fin",
)


def kernel(x, edge_index, alpha1_0, alpha2_0, gamma_0, bias_0, act_weight_0,
           alpha1_1, alpha2_1, gamma_1, bias_1):
    x0 = jnp.pad(x.reshape(N), (0, NPAD - N))
    ei_flat = edge_index.reshape(2 * E)

    s0_par, deg_par = _sweep_deg(x0, ei_flat)

    x1_2d, logd_2d = _mid(
        gamma_0, alpha1_0, alpha2_0, bias_0, act_weight_0,
        x0.reshape(R, 128),
        deg_par.reshape(2 * R, 128), s0_par.reshape(2 * R, 128))

    s1_par = _sweep(x1_2d.reshape(NPAD), ei_flat)

    out_2d = _fin(
        gamma_1, alpha1_1, alpha2_1, bias_1,
        x1_2d, logd_2d, s1_par.reshape(2 * R, 128))

    return out_2d.reshape(NPAD)[:N].reshape(N, 1)
